# baseline (speedup 1.0000x reference)
import functools

import jax
import jax.numpy as jnp
from jax import lax
from jax.experimental import pallas as pl
from jax.experimental.pallas import tpu as pltpu
from jax.experimental.pallas import tpu_sc as plsc

_C = 1250
_NC = 2
_NS = 16


@functools.lru_cache(maxsize=None)
def _make_scatter(N, E, D):
    NW = _NC * _NS
    n_chunks = E // _C
    assert E % _C == 0 and n_chunks % NW == 0
    per_tile = n_chunks // NW
    rpt = (((N + _NS - 1) // _NS) + 7) // 8 * 8
    Npad = rpt * _NS

    mesh = plsc.VectorSubcoreMesh(core_axis_name="c", subcore_axis_name="s")

    @functools.partial(
        pl.kernel,
        out_type=jax.ShapeDtypeStruct((_NC, Npad, D), jnp.float32),
        mesh=mesh,
        compiler_params=pltpu.CompilerParams(use_tc_tiling_on_sc=False),
        scratch_types=[
            pltpu.VMEM((_C,), jnp.int32),
            pltpu.VMEM((_C,), jnp.int32),
            pltpu.VMEM((_C, D), jnp.float32),
            pltpu.VMEM_SHARED((Npad, D), jnp.float32),
            pltpu.SemaphoreType.DMA,
        ],
    )
    def scatter_kernel(x_hbm, e_hbm, z_hbm, out_hbm, src_v, dst_v, rows_v, acc, sem):
        c = lax.axis_index("c")
        s = lax.axis_index("s")
        wid = c * _NS + s

        r0 = s * rpt
        pltpu.sync_copy(z_hbm, acc.at[pl.ds(r0, rpt)])
        plsc.subcore_barrier()

        def chunk_body(ci, carry):
            u = wid * per_tile + ci
            pltpu.sync_copy(e_hbm.at[0, u], src_v)
            pltpu.sync_copy(e_hbm.at[1, u], dst_v)
            pltpu.async_copy(x_hbm.at[src_v], rows_v, sem).wait()
            pltpu.sync_copy(rows_v, acc.at[dst_v], add=True)
            return carry

        lax.fori_loop(0, per_tile, chunk_body, 0)

        plsc.subcore_barrier()

        pltpu.sync_copy(acc.at[pl.ds(r0, rpt)], out_hbm.at[c, pl.ds(r0, rpt)])

    return scatter_kernel, Npad, rpt


@functools.lru_cache(maxsize=None)
def _make_combine(N, Npad, D, relu):
    BN = 2000
    assert N % BN == 0

    def body(p_ref, w_ref, b_ref, o_ref):
        sm = p_ref[0] + p_ref[1]
        h = jnp.dot(sm, w_ref[:], preferred_element_type=jnp.float32) + b_ref[:]
        o_ref[:] = jnp.maximum(h, 0.0) if relu else h

    return pl.pallas_call(
        body,
        grid=(N // BN,),
        in_specs=[
            pl.BlockSpec((2, BN, D), lambda i: (0, i, 0)),
            pl.BlockSpec((D, D), lambda i: (0, 0)),
            pl.BlockSpec((1, D), lambda i: (0, 0)),
        ],
        out_specs=pl.BlockSpec((BN, D), lambda i: (i, 0)),
        out_shape=jax.ShapeDtypeStruct((N, D), jnp.float32),
    )


def kernel(x, edge_index, W1, b1, W2, b2):
    N, D = x.shape
    E = edge_index.shape[1]
    e3 = edge_index.reshape(2, E // _C, _C)

    scatter, Npad, rpt = _make_scatter(N, E, D)
    zeros = jnp.zeros((rpt, D), jnp.float32)

    p1 = scatter(x, e3, zeros)
    h1 = _make_combine(N, Npad, D, True)(p1, W1, b1.reshape(1, D))
    p2 = scatter(h1, e3, zeros)
    out = _make_combine(N, Npad, D, False)(p2, W2, b2.reshape(1, D))
    return out


# speedup vs baseline: 1.4930x; 1.4930x over previous
import functools

import jax
import jax.numpy as jnp
from jax import lax
from jax.experimental import pallas as pl
from jax.experimental.pallas import tpu as pltpu
from jax.experimental.pallas import tpu_sc as plsc

_B = 128
_K = 5
_NC = 2
_NS = 16


@functools.lru_cache(maxsize=None)
def _make_scatter(N, E, D):
    NW = _NC * _NS
    n_units = E // (_K * _B)
    assert E % (_K * _B) == 0
    per_tile = n_units // NW // 2 * 2
    extra = n_units - per_tile * NW
    assert extra < NW
    half = per_tile // 2
    rpt = (((N + _NS - 1) // _NS) + 7) // 8 * 8
    Npad = rpt * _NS

    mesh = plsc.VectorSubcoreMesh(core_axis_name="c", subcore_axis_name="s")

    @functools.partial(
        pl.kernel,
        out_type=jax.ShapeDtypeStruct((_NC, Npad, D), jnp.float32),
        mesh=mesh,
        compiler_params=pltpu.CompilerParams(use_tc_tiling_on_sc=False),
        scratch_types=[
            pltpu.VMEM((2, _K, _B), jnp.int32),
            pltpu.VMEM((2, _K, _B), jnp.int32),
            pltpu.VMEM((2, _K, _B, D), jnp.float32),
            pltpu.VMEM_SHARED((Npad, D), jnp.float32),
            pltpu.SemaphoreType.DMA,
            pltpu.SemaphoreType.DMA,
            pltpu.SemaphoreType.DMA,
            pltpu.SemaphoreType.DMA,
            pltpu.SemaphoreType.DMA,
        ],
    )
    def scatter_kernel(x_hbm, e_hbm, z_hbm, out_hbm, src_v, dst_v, rows_v, acc,
                       sem_i, sem_g0, sem_g1, sem_s0, sem_s1):
        c = lax.axis_index("c")
        s = lax.axis_index("s")
        wid = c * _NS + s
        sem_g = (sem_g0, sem_g1)
        sem_s = (sem_s0, sem_s1)

        def fire_gathers(p, sem):
            for j in range(_K):
                pltpu.async_copy(x_hbm.at[src_v.at[p, j]], rows_v.at[p, j], sem)

        def drain_gathers(p, sem):
            for j in range(_K):
                pltpu.make_async_copy(x_hbm.at[src_v.at[p, j]], rows_v.at[p, j], sem).wait()

        def fire_scatters(p, sem):
            for j in range(_K):
                pltpu.async_copy(rows_v.at[p, j], acc.at[dst_v.at[p, j]], sem, add=True)

        def drain_scatters(p, sem):
            for j in range(_K):
                pltpu.make_async_copy(rows_v.at[p, j], acc.at[dst_v.at[p, j]], sem).wait()

        def load_idx(p, u):
            d0 = pltpu.async_copy(e_hbm.at[0, u], src_v.at[p], sem_i)
            d1 = pltpu.async_copy(e_hbm.at[1, u], dst_v.at[p], sem_i)
            return d0, d1

        r0 = s * rpt
        pltpu.sync_copy(z_hbm, acc.at[pl.ds(r0, rpt)])
        plsc.subcore_barrier()

        base = wid * per_tile

        for d in load_idx(0, base):
            d.wait()
        fire_gathers(0, sem_g0)

        def pair_body(i, carry):
            u0 = base + 2 * i
            drain_gathers(0, sem_g0)

            @pl.when(i > 0)
            def _():
                drain_scatters(1, sem_s1)

            di = load_idx(1, u0 + 1)
            fire_scatters(0, sem_s0)
            for d in di:
                d.wait()
            fire_gathers(1, sem_g1)

            drain_gathers(1, sem_g1)
            drain_scatters(0, sem_s0)
            fire_scatters(1, sem_s1)

            @pl.when(i < half - 1)
            def _():
                di2 = load_idx(0, u0 + 2)
                for d in di2:
                    d.wait()
                fire_gathers(0, sem_g0)

            return carry

        lax.fori_loop(0, half, pair_body, 0)
        drain_scatters(1, sem_s1)

        @pl.when(wid < extra)
        def _():
            for d in load_idx(0, NW * per_tile + wid):
                d.wait()
            fire_gathers(0, sem_g0)
            drain_gathers(0, sem_g0)
            fire_scatters(0, sem_s0)
            drain_scatters(0, sem_s0)

        plsc.subcore_barrier()

        pltpu.sync_copy(acc.at[pl.ds(r0, rpt)], out_hbm.at[c, pl.ds(r0, rpt)])

    return scatter_kernel, Npad, rpt


@functools.lru_cache(maxsize=None)
def _make_combine(N, Npad, D, relu):
    BN = 2000
    assert N % BN == 0

    def body(p_ref, w_ref, b_ref, o_ref):
        sm = p_ref[0] + p_ref[1]
        h = jnp.dot(sm, w_ref[:], preferred_element_type=jnp.float32) + b_ref[:]
        o_ref[:] = jnp.maximum(h, 0.0) if relu else h

    return pl.pallas_call(
        body,
        grid=(N // BN,),
        in_specs=[
            pl.BlockSpec((2, BN, D), lambda i: (0, i, 0)),
            pl.BlockSpec((D, D), lambda i: (0, 0)),
            pl.BlockSpec((1, D), lambda i: (0, 0)),
        ],
        out_specs=pl.BlockSpec((BN, D), lambda i: (i, 0)),
        out_shape=jax.ShapeDtypeStruct((N, D), jnp.float32),
    )


def kernel(x, edge_index, W1, b1, W2, b2):
    N, D = x.shape
    E = edge_index.shape[1]
    e3 = edge_index.reshape(2, E // (_K * _B), _K, _B)

    scatter, Npad, rpt = _make_scatter(N, E, D)
    zeros = jnp.zeros((rpt, D), jnp.float32)

    p1 = scatter(x, e3, zeros)
    h1 = _make_combine(N, Npad, D, True)(p1, W1, b1.reshape(1, D))
    p2 = scatter(h1, e3, zeros)
    out = _make_combine(N, Npad, D, False)(p2, W2, b2.reshape(1, D))
    return out


# speedup vs baseline: 1.4947x; 1.0011x over previous
import functools

import jax
import jax.numpy as jnp
from jax import lax
from jax.experimental import pallas as pl
from jax.experimental.pallas import tpu as pltpu
from jax.experimental.pallas import tpu_sc as plsc

_B = 128
_K = 5
_NC = 2
_NS = 16


@functools.lru_cache(maxsize=None)
def _make_scatter(N, E, D):
    NW = _NC * _NS
    n_units = E // (_K * _B)
    assert E % (_K * _B) == 0
    per_tile = n_units // NW // 2 * 2
    extra = n_units - per_tile * NW
    assert extra < NW
    half = per_tile // 2
    rpt = (((N + _NS - 1) // _NS) + 7) // 8 * 8
    Npad = rpt * _NS

    mesh = plsc.VectorSubcoreMesh(core_axis_name="c", subcore_axis_name="s")

    @functools.partial(
        pl.kernel,
        out_type=jax.ShapeDtypeStruct((_NC, Npad, D), jnp.float32),
        mesh=mesh,
        compiler_params=pltpu.CompilerParams(use_tc_tiling_on_sc=False),
        scratch_types=[
            pltpu.VMEM((2, _K, _B), jnp.int32),
            pltpu.VMEM((2, _K, _B), jnp.int32),
            pltpu.VMEM((2, _K, _B, D), jnp.float32),
            pltpu.VMEM_SHARED((Npad, D), jnp.float32),
            pltpu.SemaphoreType.DMA,
            pltpu.SemaphoreType.DMA,
            pltpu.SemaphoreType.DMA,
            pltpu.SemaphoreType.DMA,
            pltpu.SemaphoreType.DMA,
        ],
    )
    def scatter_kernel(x_hbm, e_hbm, z_hbm, out_hbm, src_v, dst_v, rows_v, acc,
                       sem_i, sem_g0, sem_g1, sem_s0, sem_s1):
        c = lax.axis_index("c")
        s = lax.axis_index("s")
        wid = c * _NS + s
        sem_g = (sem_g0, sem_g1)
        sem_s = (sem_s0, sem_s1)

        def fire_gathers(p, sem):
            for j in range(_K):
                pltpu.async_copy(x_hbm.at[src_v.at[p, j]], rows_v.at[p, j], sem)

        def drain_gathers(p, sem):
            for j in range(_K):
                pltpu.make_async_copy(x_hbm.at[src_v.at[p, j]], rows_v.at[p, j], sem).wait()

        def fire_scatters(p, sem):
            for j in range(_K):
                pltpu.async_copy(rows_v.at[p, j], acc.at[dst_v.at[p, j]], sem, add=True)

        def drain_scatters(p, sem):
            for j in range(_K):
                pltpu.make_async_copy(rows_v.at[p, j], acc.at[dst_v.at[p, j]], sem).wait()

        def load_idx(p, u):
            ds = []
            for j in range(_K):
                off = u * (_K * _B) + j * _B
                ds.append(pltpu.async_copy(e_hbm.at[0, pl.ds(off, _B)],
                                           src_v.at[p, j], sem_i))
                ds.append(pltpu.async_copy(e_hbm.at[1, pl.ds(off, _B)],
                                           dst_v.at[p, j], sem_i))
            return ds

        r0 = s * rpt
        pltpu.sync_copy(z_hbm, acc.at[pl.ds(r0, rpt)])
        plsc.subcore_barrier()

        base = wid * per_tile

        for d in load_idx(0, base):
            d.wait()
        fire_gathers(0, sem_g0)

        def pair_body(i, carry):
            u0 = base + 2 * i
            drain_gathers(0, sem_g0)

            @pl.when(i > 0)
            def _():
                drain_scatters(1, sem_s1)

            di = load_idx(1, u0 + 1)
            fire_scatters(0, sem_s0)
            for d in di:
                d.wait()
            fire_gathers(1, sem_g1)

            drain_gathers(1, sem_g1)
            drain_scatters(0, sem_s0)
            fire_scatters(1, sem_s1)

            @pl.when(i < half - 1)
            def _():
                di2 = load_idx(0, u0 + 2)
                for d in di2:
                    d.wait()
                fire_gathers(0, sem_g0)

            return carry

        lax.fori_loop(0, half, pair_body, 0)
        drain_scatters(1, sem_s1)

        @pl.when(wid < extra)
        def _():
            for d in load_idx(0, NW * per_tile + wid):
                d.wait()
            fire_gathers(0, sem_g0)
            drain_gathers(0, sem_g0)
            fire_scatters(0, sem_s0)
            drain_scatters(0, sem_s0)

        plsc.subcore_barrier()

        pltpu.sync_copy(acc.at[pl.ds(r0, rpt)], out_hbm.at[c, pl.ds(r0, rpt)])

    return scatter_kernel, Npad, rpt


@functools.lru_cache(maxsize=None)
def _make_combine(N, Npad, D, relu):
    BN = 2000
    assert N % BN == 0

    def body(p_ref, w_ref, b_ref, o_ref):
        sm = p_ref[0] + p_ref[1]
        h = jnp.dot(sm, w_ref[:], preferred_element_type=jnp.float32) + b_ref[:]
        o_ref[:] = jnp.maximum(h, 0.0) if relu else h

    return pl.pallas_call(
        body,
        grid=(N // BN,),
        in_specs=[
            pl.BlockSpec((2, BN, D), lambda i: (0, i, 0)),
            pl.BlockSpec((D, D), lambda i: (0, 0)),
            pl.BlockSpec((1, D), lambda i: (0, 0)),
        ],
        out_specs=pl.BlockSpec((BN, D), lambda i: (i, 0)),
        out_shape=jax.ShapeDtypeStruct((N, D), jnp.float32),
    )


def kernel(x, edge_index, W1, b1, W2, b2):
    N, D = x.shape
    E = edge_index.shape[1]
    e3 = edge_index

    scatter, Npad, rpt = _make_scatter(N, E, D)
    zeros = jnp.zeros((rpt, D), jnp.float32)

    p1 = scatter(x, e3, zeros)
    h1 = _make_combine(N, Npad, D, True)(p1, W1, b1.reshape(1, D))
    p2 = scatter(h1, e3, zeros)
    out = _make_combine(N, Npad, D, False)(p2, W2, b2.reshape(1, D))
    return out
